# Initial kernel scaffold; baseline (speedup 1.0000x reference)
import jax
import jax.numpy as jnp
from jax.experimental import pallas as pl


def kernel(boxes, scores):
    raise NotImplementedError("write your pallas kernel here")


# speedup vs baseline: 62.1890x; 62.1890x over previous
import jax
import jax.numpy as jnp
from jax.experimental import pallas as pl
from jax.experimental.pallas import tpu as pltpu

_N = 5000
_B = 128
_NB = 40
_P = _NB * _B
_IOU_T = 0.5


def _nms_kernel(x1c, y1c, x2c, y2c, x1r, y1r, x2r, y2r, keep_out, keepc):
    keepc[...] = jnp.zeros((_P, 1), jnp.float32)
    keep_out[...] = jnp.zeros((_NB, _B), jnp.float32)

    ax1 = x1r[...]
    ay1 = y1r[...]
    ax2 = x2r[...]
    ay2 = y2r[...]
    area_r = (ax2 - ax1) * (ay2 - ay1)

    rowg = jax.lax.broadcasted_iota(jnp.int32, (_P, _B), 0)
    rloc = jax.lax.broadcasted_iota(jnp.int32, (_B, _B), 0)
    cloc = jax.lax.broadcasted_iota(jnp.int32, (_B, _B), 1)

    def block_step(i, carry):
        bx1 = x1c[pl.ds(i, 1), :]
        by1 = y1c[pl.ds(i, 1), :]
        bx2 = x2c[pl.ds(i, 1), :]
        by2 = y2c[pl.ds(i, 1), :]
        area_c = (bx2 - bx1) * (by2 - by1)

        ix1 = jnp.maximum(ax1, bx1)
        iy1 = jnp.maximum(ay1, by1)
        ix2 = jnp.minimum(ax2, bx2)
        iy2 = jnp.minimum(ay2, by2)
        w = jnp.maximum(ix2 - ix1, 0.0)
        h = jnp.maximum(iy2 - iy1, 0.0)
        inter = w * h
        iou = inter / (area_r + area_c - inter + 1e-9)
        over = iou > _IOU_T

        cross = over & (rowg < i * _B)
        sup = jnp.max(jnp.where(cross, keepc[...], 0.0), axis=0, keepdims=True)
        free = (sup == 0.0).astype(jnp.float32)

        wx1 = x1r[pl.ds(i * _B, _B), :]
        wy1 = y1r[pl.ds(i * _B, _B), :]
        wx2 = x2r[pl.ds(i * _B, _B), :]
        wy2 = y2r[pl.ds(i * _B, _B), :]
        warea = (wx2 - wx1) * (wy2 - wy1)
        jx1 = jnp.maximum(wx1, bx1)
        jy1 = jnp.maximum(wy1, by1)
        jx2 = jnp.minimum(wx2, bx2)
        jy2 = jnp.minimum(wy2, by2)
        jw = jnp.maximum(jx2 - jx1, 0.0)
        jh = jnp.maximum(jy2 - jy1, 0.0)
        jinter = jw * jh
        jiou = jinter / (warea + area_c - jinter + 1e-9)
        tf = ((jiou > _IOU_T) & (rloc < cloc)).astype(jnp.float32)

        def fcond(c):
            return c[1]

        def fbody(c):
            k, _ = c
            s = jnp.dot(k, tf, preferred_element_type=jnp.float32)
            k2 = free * (s == 0.0).astype(jnp.float32)
            return k2, jnp.any(k2 != k)

        k0 = free
        s0 = jnp.dot(k0, tf, preferred_element_type=jnp.float32)
        k1 = free * (s0 == 0.0).astype(jnp.float32)
        k, _ = jax.lax.while_loop(fcond, fbody, (k1, jnp.any(k1 != k0)))

        keep_out[pl.ds(i, 1), :] = k
        keepc[pl.ds(i * _B, _B), :] = jnp.transpose(k, (1, 0))
        return carry

    jax.lax.fori_loop(0, _NB, block_step, 0, unroll=False)


def kernel(boxes, scores):
    pad = _P - _N
    scores_p = jnp.concatenate(
        [scores, jnp.full((pad,), -jnp.inf, dtype=jnp.float32)])
    boxes_p = jnp.concatenate(
        [boxes, jnp.zeros((pad, 4), dtype=jnp.float32)], axis=0)
    order = jnp.argsort(-scores_p)
    bs = boxes_p[order]

    comps_c = [bs[:, j].reshape(_NB, _B) for j in range(4)]
    comps_r = [bs[:, j].reshape(_P, 1) for j in range(4)]

    keep_sorted = pl.pallas_call(
        _nms_kernel,
        out_shape=jax.ShapeDtypeStruct((_NB, _B), jnp.float32),
        scratch_shapes=[pltpu.VMEM((_P, 1), jnp.float32)],
    )(*comps_c, *comps_r)

    keep = (jnp.zeros((_P,), jnp.bool_)
            .at[order].set(keep_sorted.reshape(_P) > 0.0)[:_N])
    masked = scores * keep.astype(scores.dtype)
    return masked, keep
